# baseline (speedup 1.0000x reference)
import functools

import jax
import jax.numpy as jnp
from jax import lax
from jax.experimental import pallas as pl
from jax.experimental.pallas import tpu as pltpu
from jax.experimental.pallas import tpu_sc as plsc

N = 10000
E = 320000
D = 128
DH = D // 2
ITERS = 5
K = 4000

NC, NS = 2, 16
NW = NC * NS
CH = 128

NBUF = 2
NPASS = 8
CHT = 160
CHN = CHT // NPASS
EPW = CHT * CH
EP = EPW * NS
PAD_E = EP - E

ROWS2 = N + 112
RPT = ROWS2 // NS
PCH = 128
NPIECE = -(-RPT // PCH)

CH2 = 128
KP = 4096
R1 = 10240
ACC1 = 2 * R1
SPT1 = ACC1 // NS
IDX2 = EP + KP
EPT2 = IDX2 // NW
CHUNKS2 = EPT2 // CH2

_MESH = plsc.VectorSubcoreMesh(
    core_axis_name="c", subcore_axis_name="s", num_cores=NC, num_subcores=NS
)


@functools.partial(
    pl.kernel,
    out_type=jax.ShapeDtypeStruct((NC * ACC1,), jnp.float32),
    mesh=_MESH,
    scratch_types=[
        pltpu.VMEM_SHARED((ACC1,), jnp.float32),
        pltpu.VMEM((CH2,), jnp.float32),
        pltpu.VMEM((CH2,), jnp.int32),
    ],
)
def _sc_histograms(idx_hbm, zeros1_hbm, out_hbm, acc_sh, ones_v, idx_v):
    c = lax.axis_index("c")
    s = lax.axis_index("s")
    wid = s * NC + c
    pltpu.sync_copy(
        zeros1_hbm.at[pl.ds(s * SPT1, SPT1)], acc_sh.at[pl.ds(s * SPT1, SPT1)]
    )
    for i in range(CH2 // 16):
        ones_v[pl.ds(i * 16, 16)] = jnp.full((16,), 1.0, jnp.float32)
    plsc.subcore_barrier()

    def body(j, carry):
        off = wid * EPT2 + j * CH2
        pltpu.sync_copy(idx_hbm.at[pl.ds(off, CH2)], idx_v)
        pltpu.sync_copy(ones_v, acc_sh.at[idx_v], add=True)
        return carry

    lax.fori_loop(0, CHUNKS2, body, 0)
    plsc.subcore_barrier()
    pltpu.sync_copy(
        acc_sh.at[pl.ds(s * SPT1, SPT1)],
        out_hbm.at[pl.ds(c * ACC1 + s * SPT1, SPT1)],
    )


@functools.partial(
    pl.kernel,
    out_type=jax.ShapeDtypeStruct((NC * ROWS2, DH), jnp.float32),
    mesh=_MESH,
    scratch_types=[
        pltpu.VMEM_SHARED((ROWS2, DH), jnp.float32),
        pltpu.VMEM_SHARED((ROWS2, DH), jnp.float32),
        pltpu.VMEM((CHN, CH), jnp.int32),
        pltpu.VMEM((CHN, CH), jnp.int32),
    ]
    + [pltpu.VMEM((CH, DH), jnp.float32)] * NBUF
    + [pltpu.SemaphoreType.DMA] * (2 * NBUF),
)
def _sc_spmm(y2_hbm, colp_hbm, rowp_hbm, out_hbm,
             y_sh, acc_sh, cidx_all, ridx_all, b0, b1,
             g0, g1, s0, s1):
    bufs = [b0, b1]
    gsems = [g0, g1]
    ssems = [s0, s1]
    c = lax.axis_index("c")
    s = lax.axis_index("s")
    row0 = s * RPT

    z16 = jnp.zeros((16,), jnp.float32)
    for r in range(CH):
        for k in range(DH // 16):
            b0[r, pl.ds(k * 16, 16)] = z16
    for p in range(NPIECE):
        nr = min(PCH, RPT - p * PCH)
        pltpu.async_copy(b0.at[pl.ds(0, nr)],
                         acc_sh.at[pl.ds(row0 + p * PCH, nr)], g0)
    for p in range(NPIECE):
        nr = min(PCH, RPT - p * PCH)
        pltpu.make_async_copy(b0.at[pl.ds(0, nr)],
                              acc_sh.at[pl.ds(row0 + p * PCH, nr)], g0).wait()
    for p in range(NPIECE):
        nr = min(PCH, RPT - p * PCH)
        b = bufs[p % 2]
        sem = gsems[p % 2]
        if p >= 2:
            prev_nr = min(PCH, RPT - (p - 2) * PCH)
            pltpu.make_async_copy(
                b.at[pl.ds(0, prev_nr)],
                y_sh.at[pl.ds(row0 + (p - 2) * PCH, prev_nr)], sem).wait()
        pltpu.sync_copy(
            y2_hbm.at[pl.ds(c * ROWS2 + row0 + p * PCH, nr)],
            b.at[pl.ds(0, nr)])
        pltpu.async_copy(b.at[pl.ds(0, nr)],
                         y_sh.at[pl.ds(row0 + p * PCH, nr)], sem)
    for p in range(max(0, NPIECE - 2), NPIECE):
        nr = min(PCH, RPT - p * PCH)
        pltpu.make_async_copy(bufs[p % 2].at[pl.ds(0, nr)],
                              y_sh.at[pl.ds(row0 + p * PCH, nr)],
                              gsems[p % 2]).wait()
    plsc.subcore_barrier()

    for h in range(NPASS):
        pltpu.sync_copy(colp_hbm.at[s, h], cidx_all)
        pltpu.sync_copy(rowp_hbm.at[s, h], ridx_all)
        for b in range(NBUF):
            pltpu.async_copy(y_sh.at[cidx_all.at[b]], bufs[b], gsems[b])

        def round_body(r, carry):
            for b in range(NBUF):
                j = r * NBUF + b
                bp = (b - 1) % NBUF
                pltpu.make_async_copy(y_sh.at[cidx_all.at[j]], bufs[b],
                                      gsems[b]).wait()
                pltpu.async_copy(bufs[b], acc_sh.at[ridx_all.at[j]], ssems[b],
                                 add=True)
                @pl.when(j >= 1)
                def _():
                    pltpu.make_async_copy(bufs[bp],
                                          acc_sh.at[ridx_all.at[j - 1]],
                                          ssems[bp]).wait()

                    @pl.when(j - 1 + NBUF < CHN)
                    def _():
                        pltpu.async_copy(y_sh.at[cidx_all.at[j - 1 + NBUF]],
                                         bufs[bp], gsems[bp])
            return carry

        lax.fori_loop(0, CHN // NBUF, round_body, 0)
        last = (CHN - 1) % NBUF
        pltpu.make_async_copy(bufs[last], acc_sh.at[ridx_all.at[CHN - 1]],
                              ssems[last]).wait()
    plsc.subcore_barrier()

    out0 = c * ROWS2 + row0
    for p in range(NPIECE):
        nr = min(PCH, RPT - p * PCH)
        b = bufs[p % NBUF]
        if p >= NBUF:
            prev_nr = min(PCH, RPT - (p - NBUF) * PCH)
            pltpu.make_async_copy(
                b.at[pl.ds(0, prev_nr)],
                out_hbm.at[pl.ds(out0 + (p - NBUF) * PCH, prev_nr)],
                ssems[p % NBUF]).wait()
        pltpu.sync_copy(acc_sh.at[pl.ds(row0 + p * PCH, nr)],
                        b.at[pl.ds(0, nr)])
        pltpu.async_copy(b.at[pl.ds(0, nr)],
                         out_hbm.at[pl.ds(out0 + p * PCH, nr)],
                         ssems[p % NBUF])
    for p in range(max(0, NPIECE - NBUF), NPIECE):
        nr = min(PCH, RPT - p * PCH)
        pltpu.make_async_copy(bufs[p % NBUF].at[pl.ds(0, nr)],
                              out_hbm.at[pl.ds(out0 + p * PCH, nr)],
                              ssems[p % NBUF]).wait()


def _remap_body(r_ref, c_ref, o_ref):
    r = r_ref[...]
    cc = c_ref[...]
    lane = lax.broadcasted_iota(jnp.int32, r.shape, 1)
    o_ref[...] = jnp.where(r == cc, N + (lane % 16), r)


def _tc_remap(row2d, col2d):
    return pl.pallas_call(
        _remap_body,
        out_shape=jax.ShapeDtypeStruct(row2d.shape, jnp.int32),
    )(row2d, col2d)


_BLK = 1264
_NBLK = ROWS2 // _BLK


def _prep_body(d0_ref, d1_ref, k0_ref, k1_ref, x_ref,
               g_ref, disb_ref, xk_ref, y2_ref):
    deg = d0_ref[...] + d1_ref[...]
    dis = jnp.where(deg > 0.0, lax.rsqrt(deg), 0.0)
    m = ((k0_ref[...] + k1_ref[...]) > 0.0).astype(jnp.float32)
    disb = jnp.broadcast_to(dis, x_ref.shape)
    xk = x_ref[...] * m
    g_ref[...] = disb * (1.0 - m)
    disb_ref[...] = disb
    xk_ref[...] = xk
    y2_ref[...] = disb * xk


def _tc_prep(d0, d1, k0, k1, xpad2):
    col_spec = pl.BlockSpec((_BLK, 1), lambda h, i: (i, 0))
    flat_spec = pl.BlockSpec((_BLK, DH), lambda h, i: (h * _NBLK + i, 0))
    out_sd = jax.ShapeDtypeStruct((NC * ROWS2, DH), jnp.float32)
    return pl.pallas_call(
        _prep_body,
        grid=(NC, _NBLK),
        in_specs=[col_spec] * 4 + [flat_spec],
        out_specs=[flat_spec] * 4,
        out_shape=[out_sd] * 4,
    )(d0, d1, k0, k1, xpad2)


def _combine_body(raw_ref, g_ref, disb_ref, xk_ref, y2_ref, ob_ref):
    ob = g_ref[...] * raw_ref[...] + xk_ref[...]
    y2_ref[...] = disb_ref[...] * ob
    ob_ref[...] = ob


def _tc_combine(rawflat, g, disb, xk):
    flat_spec = pl.BlockSpec((_BLK, DH), lambda h, i: (h * _NBLK + i, 0))
    out_sd = jax.ShapeDtypeStruct((NC * ROWS2, DH), jnp.float32)
    return pl.pallas_call(
        _combine_body,
        grid=(NC, _NBLK),
        in_specs=[flat_spec] * 4,
        out_specs=[flat_spec] * 2,
        out_shape=[out_sd] * 2,
    )(rawflat, g, disb, xk)


def kernel(x, edge_index, known_feature_mask):
    row = edge_index[0]
    col = edge_index[1]

    pad_i = jnp.arange(PAD_E, dtype=jnp.int32)
    row_pad = jnp.concatenate([row, N + (pad_i % 16)])
    col_pad = jnp.concatenate([col, pad_i % N])
    kpad_i = jnp.arange(KP - K, dtype=jnp.int32)
    known_pad = jnp.concatenate(
        [known_feature_mask + R1, R1 + N + (kpad_i % 16)]
    )
    hist_idx = jnp.concatenate([row_pad, known_pad])
    xpad = jnp.concatenate([x, jnp.zeros((ROWS2 - N, D), jnp.float32)])
    xpad2 = jnp.concatenate([xpad[:, :DH], xpad[:, DH:]], axis=0)

    zeros1 = jnp.zeros((ACC1,), jnp.float32)

    hist = _sc_histograms(hist_idx, zeros1)
    d0 = hist[0:ROWS2].reshape(ROWS2, 1)
    d1 = hist[ACC1:ACC1 + ROWS2].reshape(ROWS2, 1)
    k0 = hist[R1:R1 + ROWS2].reshape(ROWS2, 1)
    k1 = hist[ACC1 + R1:ACC1 + R1 + ROWS2].reshape(ROWS2, 1)

    rowp = _tc_remap(
        row_pad.reshape(EP // D, D), col_pad.reshape(EP // D, D)
    ).reshape(EP)

    g, disb, xk, y2 = _tc_prep(d0, d1, k0, k1, xpad2)

    colp4 = col_pad.reshape(NS, NPASS, CHN, CH)
    rowp4 = rowp.reshape(NS, NPASS, CHN, CH)
    ob = None
    for _ in range(ITERS):
        rawflat = _sc_spmm(y2, colp4, rowp4)
        y2, ob = _tc_combine(rawflat, g, disb, xk)
    return jnp.concatenate([ob[0:N], ob[ROWS2:ROWS2 + N]], axis=1)


# speedup vs baseline: 1.4081x; 1.4081x over previous
import functools

import jax
import jax.numpy as jnp
from jax import lax
from jax.experimental import pallas as pl
from jax.experimental.pallas import tpu as pltpu
from jax.experimental.pallas import tpu_sc as plsc

N = 10000
E = 320000
D = 128
ITERS = 5
K = 4000

NC, NS = 2, 16
NW = NC * NS
CH = 32

NBUF = 8
NPASS = 8
CHUNKS = 320
EPT = CHUNKS * CH
EP = EPT * NW
PAD_E = EP - E

ROWS = N + 112
RPT = ROWS // NS

CH2 = 128
KP = 4096
R1 = 10240
ACC1 = 2 * R1
SPT1 = ACC1 // NS
IDX2 = EP + KP
EPT2 = IDX2 // NW
CHUNKS2 = EPT2 // CH2

_MESH = plsc.VectorSubcoreMesh(
    core_axis_name="c", subcore_axis_name="s", num_cores=NC, num_subcores=NS
)


@functools.partial(
    pl.kernel,
    out_type=jax.ShapeDtypeStruct((NC * ACC1,), jnp.float32),
    mesh=_MESH,
    scratch_types=[
        pltpu.VMEM_SHARED((ACC1,), jnp.float32),
        pltpu.VMEM((CH2,), jnp.float32),
        pltpu.VMEM((CH2,), jnp.int32),
    ],
)
def _sc_histograms(idx_hbm, zeros1_hbm, out_hbm, acc_sh, ones_v, idx_v):
    c = lax.axis_index("c")
    s = lax.axis_index("s")
    wid = s * NC + c
    pltpu.sync_copy(
        zeros1_hbm.at[pl.ds(s * SPT1, SPT1)], acc_sh.at[pl.ds(s * SPT1, SPT1)]
    )
    for i in range(CH2 // 16):
        ones_v[pl.ds(i * 16, 16)] = jnp.full((16,), 1.0, jnp.float32)
    plsc.subcore_barrier()

    def body(j, carry):
        off = wid * EPT2 + j * CH2
        pltpu.sync_copy(idx_hbm.at[pl.ds(off, CH2)], idx_v)
        pltpu.sync_copy(ones_v, acc_sh.at[idx_v], add=True)
        return carry

    lax.fori_loop(0, CHUNKS2, body, 0)
    plsc.subcore_barrier()
    pltpu.sync_copy(
        acc_sh.at[pl.ds(s * SPT1, SPT1)],
        out_hbm.at[pl.ds(c * ACC1 + s * SPT1, SPT1)],
    )


@functools.partial(
    pl.kernel,
    out_type=jax.ShapeDtypeStruct((NC * ROWS, D), jnp.float32),
    mesh=_MESH,
    scratch_types=[
        pltpu.VMEM_SHARED((ROWS, D), jnp.float32),
        pltpu.VMEM((CHUNKS // NPASS, CH), jnp.int32),
        pltpu.VMEM((CHUNKS // NPASS, CH), jnp.int32),
    ]
    + [pltpu.VMEM((CH, D), jnp.float32)] * NBUF
    + [pltpu.SemaphoreType.DMA] * (2 * NBUF),
)
def _sc_spmm(y_hbm, colp_hbm, rowp_hbm, out_hbm,
             acc_sh, cidx_all, ridx_all, b0, b1, b2, b3, b4, b5, b6, b7,
             g0, g1, g2, g3, g4, g5, g6, g7,
             s0, s1, s2, s3, s4, s5, s6, s7):
    bufs = [b0, b1, b2, b3, b4, b5, b6, b7]
    gsems = [g0, g1, g2, g3, g4, g5, g6, g7]
    ssems = [s0, s1, s2, s3, s4, s5, s6, s7]
    c = lax.axis_index("c")
    s = lax.axis_index("s")
    wid = s * NC + c
    z16 = jnp.zeros((16,), jnp.float32)
    for r in range(CH):
        for k in range(D // 16):
            b0[r, pl.ds(k * 16, 16)] = z16
    row0 = s * RPT
    for p in range(RPT // CH):
        pltpu.async_copy(b0, acc_sh.at[pl.ds(row0 + p * CH, CH)], g0)
    tail = RPT % CH
    if tail:
        pltpu.async_copy(b0.at[pl.ds(0, tail)],
                         acc_sh.at[pl.ds(row0 + RPT - tail, tail)], g0)
    for p in range(RPT // CH):
        pltpu.make_async_copy(b0, acc_sh.at[pl.ds(row0 + p * CH, CH)],
                              g0).wait()
    if tail:
        pltpu.make_async_copy(b0.at[pl.ds(0, tail)],
                              acc_sh.at[pl.ds(row0 + RPT - tail, tail)],
                              g0).wait()
    plsc.subcore_barrier()

    HC = CHUNKS // NPASS
    for h in range(NPASS):
        pltpu.sync_copy(colp_hbm.at[wid, h], cidx_all)
        pltpu.sync_copy(rowp_hbm.at[wid, h], ridx_all)
        for b in range(NBUF):
            pltpu.async_copy(y_hbm.at[cidx_all.at[b]], bufs[b], gsems[b])

        def round_body(r, carry):
            for b in range(NBUF):
                j = r * NBUF + b
                bp = (b - 1) % NBUF
                pltpu.make_async_copy(y_hbm.at[cidx_all.at[j]], bufs[b],
                                      gsems[b]).wait()
                pltpu.async_copy(bufs[b], acc_sh.at[ridx_all.at[j]], ssems[b],
                                 add=True)
                @pl.when(j >= 1)
                def _():
                    pltpu.make_async_copy(bufs[bp],
                                          acc_sh.at[ridx_all.at[j - 1]],
                                          ssems[bp]).wait()

                    @pl.when(j - 1 + NBUF < HC)
                    def _():
                        pltpu.async_copy(y_hbm.at[cidx_all.at[j - 1 + NBUF]],
                                         bufs[bp], gsems[bp])
            return carry

        lax.fori_loop(0, HC // NBUF, round_body, 0)
        last = (HC - 1) % NBUF
        pltpu.make_async_copy(bufs[last], acc_sh.at[ridx_all.at[HC - 1]],
                              ssems[last]).wait()
    plsc.subcore_barrier()
    out0 = c * ROWS + s * RPT
    npiece = -(-RPT // CH)
    for p in range(npiece):
        nr = min(CH, RPT - p * CH)
        b = bufs[p % NBUF]
        if p >= NBUF:
            prev_nr = min(CH, RPT - (p - NBUF) * CH)
            pltpu.make_async_copy(
                b.at[pl.ds(0, prev_nr)],
                out_hbm.at[pl.ds(out0 + (p - NBUF) * CH, prev_nr)],
                ssems[p % NBUF]).wait()
        pltpu.sync_copy(acc_sh.at[pl.ds(row0 + p * CH, nr)],
                        b.at[pl.ds(0, nr)])
        pltpu.async_copy(b.at[pl.ds(0, nr)],
                         out_hbm.at[pl.ds(out0 + p * CH, nr)],
                         ssems[p % NBUF])
    for p in range(max(0, npiece - NBUF), npiece):
        nr = min(CH, RPT - p * CH)
        pltpu.make_async_copy(bufs[p % NBUF].at[pl.ds(0, nr)],
                              out_hbm.at[pl.ds(out0 + p * CH, nr)],
                              ssems[p % NBUF]).wait()


def _remap_body(r_ref, c_ref, o_ref):
    r = r_ref[...]
    cc = c_ref[...]
    lane = lax.broadcasted_iota(jnp.int32, r.shape, 1)
    o_ref[...] = jnp.where(r == cc, N + (lane % 16), r)


def _tc_remap(row2d, col2d):
    return pl.pallas_call(
        _remap_body,
        out_shape=jax.ShapeDtypeStruct(row2d.shape, jnp.int32),
    )(row2d, col2d)


_BLK = 2000


def _prep_body(d0_ref, d1_ref, k0_ref, k1_ref, x_ref,
               g_ref, disb_ref, xk_ref, y0_ref):
    deg = d0_ref[...] + d1_ref[...]
    dis = jnp.where(deg > 0.0, lax.rsqrt(deg), 0.0)
    m = ((k0_ref[...] + k1_ref[...]) > 0.0).astype(jnp.float32)
    disb = jnp.broadcast_to(dis, x_ref.shape)
    xk = x_ref[...] * m
    g_ref[...] = disb * (1.0 - m)
    disb_ref[...] = disb
    xk_ref[...] = xk
    y0_ref[...] = disb * xk


def _tc_prep(d0, d1, k0, k1, x):
    col_spec = pl.BlockSpec((_BLK, 1), lambda i: (i, 0))
    mat_spec = pl.BlockSpec((_BLK, D), lambda i: (i, 0))
    out_sd = jax.ShapeDtypeStruct((N, D), jnp.float32)
    return pl.pallas_call(
        _prep_body,
        grid=(N // _BLK,),
        in_specs=[col_spec, col_spec, col_spec, col_spec, mat_spec],
        out_specs=[mat_spec] * 4,
        out_shape=[out_sd] * 4,
    )(d0, d1, k0, k1, x)


def _combine_body(p0_ref, p1_ref, g_ref, disb_ref, xk_ref, y_ref, ob_ref):
    raw = p0_ref[...] + p1_ref[...]
    ob = g_ref[...] * raw + xk_ref[...]
    y_ref[...] = disb_ref[...] * ob
    ob_ref[...] = ob


def _tc_combine(p0, p1, g, disb, xk):
    mat_spec = pl.BlockSpec((_BLK, D), lambda i: (i, 0))
    out_sd = jax.ShapeDtypeStruct((N, D), jnp.float32)
    return pl.pallas_call(
        _combine_body,
        grid=(N // _BLK,),
        in_specs=[mat_spec] * 5,
        out_specs=[mat_spec] * 2,
        out_shape=[out_sd] * 2,
    )(p0, p1, g, disb, xk)


def kernel(x, edge_index, known_feature_mask):
    row = edge_index[0]
    col = edge_index[1]

    pad_i = jnp.arange(PAD_E, dtype=jnp.int32)
    row_pad = jnp.concatenate([row, N + (pad_i % 16)])
    col_pad = jnp.concatenate([col, pad_i % N])
    kpad_i = jnp.arange(KP - K, dtype=jnp.int32)
    known_pad = jnp.concatenate(
        [known_feature_mask + R1, R1 + N + (kpad_i % 16)]
    )
    hist_idx = jnp.concatenate([row_pad, known_pad])

    zeros1 = jnp.zeros((ACC1,), jnp.float32)

    hist = _sc_histograms(hist_idx, zeros1)
    d0 = hist[0:N].reshape(N, 1)
    d1 = hist[ACC1:ACC1 + N].reshape(N, 1)
    k0 = hist[R1:R1 + N].reshape(N, 1)
    k1 = hist[ACC1 + R1:ACC1 + R1 + N].reshape(N, 1)

    rowp = _tc_remap(
        row_pad.reshape(EP // D, D), col_pad.reshape(EP // D, D)
    ).reshape(EP)

    g, disb, xk, y = _tc_prep(d0, d1, k0, k1, x)

    colp3 = col_pad.reshape(NW, NPASS, CHUNKS // NPASS, CH)
    rowp3 = rowp.reshape(NW, NPASS, CHUNKS // NPASS, CH)
    out = xk
    for _ in range(ITERS):
        part = _sc_spmm(y, colp3, rowp3)
        p0 = part[0:N]
        p1 = part[ROWS:ROWS + N]
        y, out = _tc_combine(p0, p1, g, disb, xk)
    return out


# speedup vs baseline: 1.5569x; 1.1056x over previous
import functools

import jax
import jax.numpy as jnp
from jax import lax
from jax.experimental import pallas as pl
from jax.experimental.pallas import tpu as pltpu
from jax.experimental.pallas import tpu_sc as plsc

N = 10000
E = 320000
D = 128
ITERS = 5
K = 4000

NC, NS = 2, 16
NW = NC * NS
CH = 64

NBUF = 4
NPASS = 4
CHUNKS = 160
EPT = CHUNKS * CH
EP = EPT * NW
PAD_E = EP - E

ROWS = N + 112
RPT = ROWS // NS

CH2 = 128
NBUF2 = 4
KP = 16384
R1 = 10240
ACC1 = 2 * R1
SPT1 = ACC1 // NS
IDX2 = EP + KP
EPT2 = IDX2 // NW
CHUNKS2 = EPT2 // CH2

_MESH = plsc.VectorSubcoreMesh(
    core_axis_name="c", subcore_axis_name="s", num_cores=NC, num_subcores=NS
)


@functools.partial(
    pl.kernel,
    out_type=jax.ShapeDtypeStruct((NC * ACC1,), jnp.float32),
    mesh=_MESH,
    scratch_types=[
        pltpu.VMEM_SHARED((ACC1,), jnp.float32),
        pltpu.VMEM((CH2,), jnp.float32),
    ]
    + [pltpu.VMEM((CH2,), jnp.int32)] * NBUF2
    + [pltpu.SemaphoreType.DMA] * (2 * NBUF2),
)
def _sc_histograms(idx_hbm, zeros1_hbm, out_hbm, acc_sh, ones_v,
                   i0, i1, i2, i3, gi0, gi1, gi2, gi3, si0, si1, si2, si3):
    ibufs = [i0, i1, i2, i3]
    isems = [gi0, gi1, gi2, gi3]
    ssems = [si0, si1, si2, si3]
    c = lax.axis_index("c")
    s = lax.axis_index("s")
    wid = s * NC + c
    base = wid * EPT2
    pltpu.sync_copy(
        zeros1_hbm.at[pl.ds(s * SPT1, SPT1)], acc_sh.at[pl.ds(s * SPT1, SPT1)]
    )
    for i in range(CH2 // 16):
        ones_v[pl.ds(i * 16, 16)] = jnp.full((16,), 1.0, jnp.float32)
    plsc.subcore_barrier()

    for b in range(NBUF2):
        pltpu.async_copy(idx_hbm.at[pl.ds(base + b * CH2, CH2)], ibufs[b],
                         isems[b])

    def body(r, carry):
        for b in range(NBUF2):
            j = r * NBUF2 + b
            bp = (b - 1) % NBUF2
            pltpu.make_async_copy(idx_hbm.at[pl.ds(base + j * CH2, CH2)],
                                  ibufs[b], isems[b]).wait()
            pltpu.async_copy(ones_v, acc_sh.at[ibufs[b]], ssems[b], add=True)

            @pl.when(j >= 1)
            def _():
                pltpu.make_async_copy(ones_v, acc_sh.at[ibufs[bp]],
                                      ssems[bp]).wait()

                @pl.when(j - 1 + NBUF2 < CHUNKS2)
                def _():
                    pltpu.async_copy(
                        idx_hbm.at[pl.ds(base + (j - 1 + NBUF2) * CH2, CH2)],
                        ibufs[bp], isems[bp])
        return carry

    lax.fori_loop(0, CHUNKS2 // NBUF2, body, 0)
    last = (CHUNKS2 - 1) % NBUF2
    pltpu.make_async_copy(ones_v, acc_sh.at[ibufs[last]], ssems[last]).wait()
    plsc.subcore_barrier()
    pltpu.sync_copy(
        acc_sh.at[pl.ds(s * SPT1, SPT1)],
        out_hbm.at[pl.ds(c * ACC1 + s * SPT1, SPT1)],
    )


@functools.partial(
    pl.kernel,
    out_type=jax.ShapeDtypeStruct((NC * ROWS, D), jnp.float32),
    mesh=_MESH,
    scratch_types=[
        pltpu.VMEM_SHARED((ROWS, D), jnp.float32),
        pltpu.VMEM((CHUNKS // NPASS, CH), jnp.int32),
        pltpu.VMEM((CHUNKS // NPASS, CH), jnp.int32),
    ]
    + [pltpu.VMEM((CH, D), jnp.float32)] * NBUF
    + [pltpu.SemaphoreType.DMA] * (2 * NBUF),
)
def _sc_spmm(y_hbm, colp_hbm, rowp_hbm, out_hbm,
             acc_sh, cidx_all, ridx_all, b0, b1, b2, b3,
             g0, g1, g2, g3, s0, s1, s2, s3):
    bufs = [b0, b1, b2, b3]
    gsems = [g0, g1, g2, g3]
    ssems = [s0, s1, s2, s3]
    c = lax.axis_index("c")
    s = lax.axis_index("s")
    wid = s * NC + c
    z16 = jnp.zeros((16,), jnp.float32)
    for r in range(CH):
        for k in range(D // 16):
            b0[r, pl.ds(k * 16, 16)] = z16
    row0 = s * RPT
    for p in range(RPT // CH):
        pltpu.async_copy(b0, acc_sh.at[pl.ds(row0 + p * CH, CH)], g0)
    tail = RPT % CH
    if tail:
        pltpu.async_copy(b0.at[pl.ds(0, tail)],
                         acc_sh.at[pl.ds(row0 + RPT - tail, tail)], g0)
    for p in range(RPT // CH):
        pltpu.make_async_copy(b0, acc_sh.at[pl.ds(row0 + p * CH, CH)],
                              g0).wait()
    if tail:
        pltpu.make_async_copy(b0.at[pl.ds(0, tail)],
                              acc_sh.at[pl.ds(row0 + RPT - tail, tail)],
                              g0).wait()
    plsc.subcore_barrier()

    HC = CHUNKS // NPASS
    for h in range(NPASS):
        pltpu.sync_copy(colp_hbm.at[wid, h], cidx_all)
        pltpu.sync_copy(rowp_hbm.at[wid, h], ridx_all)
        for b in range(NBUF):
            pltpu.async_copy(y_hbm.at[cidx_all.at[b]], bufs[b], gsems[b])

        def round_body(r, carry):
            for b in range(NBUF):
                j = r * NBUF + b
                bp = (b - 1) % NBUF
                pltpu.make_async_copy(y_hbm.at[cidx_all.at[j]], bufs[b],
                                      gsems[b]).wait()
                pltpu.async_copy(bufs[b], acc_sh.at[ridx_all.at[j]], ssems[b],
                                 add=True)
                @pl.when(j >= 1)
                def _():
                    pltpu.make_async_copy(bufs[bp],
                                          acc_sh.at[ridx_all.at[j - 1]],
                                          ssems[bp]).wait()

                    @pl.when(j - 1 + NBUF < HC)
                    def _():
                        pltpu.async_copy(y_hbm.at[cidx_all.at[j - 1 + NBUF]],
                                         bufs[bp], gsems[bp])
            return carry

        lax.fori_loop(0, HC // NBUF, round_body, 0)
        last = (HC - 1) % NBUF
        pltpu.make_async_copy(bufs[last], acc_sh.at[ridx_all.at[HC - 1]],
                              ssems[last]).wait()
    plsc.subcore_barrier()
    out0 = c * ROWS + s * RPT
    npiece = -(-RPT // CH)
    for p in range(npiece):
        nr = min(CH, RPT - p * CH)
        b = bufs[p % NBUF]
        if p >= NBUF:
            prev_nr = min(CH, RPT - (p - NBUF) * CH)
            pltpu.make_async_copy(
                b.at[pl.ds(0, prev_nr)],
                out_hbm.at[pl.ds(out0 + (p - NBUF) * CH, prev_nr)],
                ssems[p % NBUF]).wait()
        pltpu.sync_copy(acc_sh.at[pl.ds(row0 + p * CH, nr)],
                        b.at[pl.ds(0, nr)])
        pltpu.async_copy(b.at[pl.ds(0, nr)],
                         out_hbm.at[pl.ds(out0 + p * CH, nr)],
                         ssems[p % NBUF])
    for p in range(max(0, npiece - NBUF), npiece):
        nr = min(CH, RPT - p * CH)
        pltpu.make_async_copy(bufs[p % NBUF].at[pl.ds(0, nr)],
                              out_hbm.at[pl.ds(out0 + p * CH, nr)],
                              ssems[p % NBUF]).wait()


def _remap_body(r_ref, c_ref, o_ref):
    r = r_ref[...]
    cc = c_ref[...]
    lane = lax.broadcasted_iota(jnp.int32, r.shape, 1)
    o_ref[...] = jnp.where(r == cc, N + (lane % 16), r)


def _tc_remap(row2d, col2d):
    return pl.pallas_call(
        _remap_body,
        out_shape=jax.ShapeDtypeStruct(row2d.shape, jnp.int32),
    )(row2d, col2d)


_BLK = 1264
_NBLK = ROWS // _BLK


def _prep_body(d0_ref, d1_ref, k0_ref, k1_ref, x_ref,
               g2_ref, xk2_ref, g_ref, xk_ref, y0_ref):
    deg = d0_ref[...] + d1_ref[...]
    dis = jnp.where(deg > 0.0, lax.rsqrt(deg), 0.0)
    m = ((k0_ref[...] + k1_ref[...]) > 0.0).astype(jnp.float32)
    disb = jnp.broadcast_to(dis, x_ref.shape)
    g = disb * (1.0 - m)
    xk = x_ref[...] * m
    g2_ref[...] = disb * g
    xk2_ref[...] = disb * xk
    g_ref[...] = g
    xk_ref[...] = xk
    y0_ref[...] = disb * xk


def _tc_prep(d0, d1, k0, k1, xpad):
    col_spec = pl.BlockSpec((_BLK, 1), lambda i: (i, 0))
    mat_spec = pl.BlockSpec((_BLK, D), lambda i: (i, 0))
    out_sd = jax.ShapeDtypeStruct((ROWS, D), jnp.float32)
    return pl.pallas_call(
        _prep_body,
        grid=(_NBLK,),
        in_specs=[col_spec, col_spec, col_spec, col_spec, mat_spec],
        out_specs=[mat_spec] * 5,
        out_shape=[out_sd] * 5,
    )(d0, d1, k0, k1, xpad)


def _mid_body(p0_ref, p1_ref, g2_ref, xk2_ref, y_ref):
    y_ref[...] = g2_ref[...] * (p0_ref[...] + p1_ref[...]) + xk2_ref[...]


def _tc_combine_mid(part, g2, xk2):
    p0_spec = pl.BlockSpec((_BLK, D), lambda i: (i, 0))
    p1_spec = pl.BlockSpec((_BLK, D), lambda i: (_NBLK + i, 0))
    mat_spec = pl.BlockSpec((_BLK, D), lambda i: (i, 0))
    return pl.pallas_call(
        _mid_body,
        grid=(_NBLK,),
        in_specs=[p0_spec, p1_spec, mat_spec, mat_spec],
        out_specs=mat_spec,
        out_shape=jax.ShapeDtypeStruct((ROWS, D), jnp.float32),
    )(part, part, g2, xk2)


def _last_body(p0_ref, p1_ref, g_ref, xk_ref, ob_ref):
    ob_ref[...] = g_ref[...] * (p0_ref[...] + p1_ref[...]) + xk_ref[...]


def _tc_combine_last(part, g, xk):
    p0_spec = pl.BlockSpec((_BLK, D), lambda i: (i, 0))
    p1_spec = pl.BlockSpec((_BLK, D), lambda i: (_NBLK + i, 0))
    mat_spec = pl.BlockSpec((_BLK, D), lambda i: (i, 0))
    return pl.pallas_call(
        _last_body,
        grid=(_NBLK,),
        in_specs=[p0_spec, p1_spec, mat_spec, mat_spec],
        out_specs=mat_spec,
        out_shape=jax.ShapeDtypeStruct((ROWS, D), jnp.float32),
    )(part, part, g, xk)


def kernel(x, edge_index, known_feature_mask):
    row = edge_index[0]
    col = edge_index[1]

    pad_i = jnp.arange(PAD_E, dtype=jnp.int32)
    row_pad = jnp.concatenate([row, N + (pad_i % 16)])
    col_pad = jnp.concatenate([col, pad_i % N])
    kpad_i = jnp.arange(KP - K, dtype=jnp.int32)
    known_pad = jnp.concatenate(
        [known_feature_mask + R1, R1 + N + (kpad_i % 16)]
    )
    hist_idx = jnp.concatenate([row_pad, known_pad])
    xpad = jnp.concatenate([x, jnp.zeros((ROWS - N, D), jnp.float32)])

    zeros1 = jnp.zeros((ACC1,), jnp.float32)

    hist = _sc_histograms(hist_idx, zeros1)
    d0 = hist[0:ROWS].reshape(ROWS, 1)
    d1 = hist[ACC1:ACC1 + ROWS].reshape(ROWS, 1)
    k0 = hist[R1:R1 + ROWS].reshape(ROWS, 1)
    k1 = hist[ACC1 + R1:ACC1 + R1 + ROWS].reshape(ROWS, 1)

    rowp = _tc_remap(
        row_pad.reshape(EP // D, D), col_pad.reshape(EP // D, D)
    ).reshape(EP)

    g2, xk2, g, xk, y = _tc_prep(d0, d1, k0, k1, xpad)

    colp3 = col_pad.reshape(NW, NPASS, CHUNKS // NPASS, CH)
    rowp3 = rowp.reshape(NW, NPASS, CHUNKS // NPASS, CH)
    for _ in range(ITERS - 1):
        part = _sc_spmm(y, colp3, rowp3)
        y = _tc_combine_mid(part, g2, xk2)
    part = _sc_spmm(y, colp3, rowp3)
    ob = _tc_combine_last(part, g, xk)
    return ob[0:N]


# speedup vs baseline: 1.5752x; 1.0118x over previous
import functools

import jax
import jax.numpy as jnp
from jax import lax
from jax.experimental import pallas as pl
from jax.experimental.pallas import tpu as pltpu
from jax.experimental.pallas import tpu_sc as plsc

N = 10000
E = 320000
D = 128
ITERS = 5
K = 4000

NC, NS = 2, 16
NW = NC * NS
CH = 64

NBUF = 4
NPASS = 4
CHUNKS = 160
EPT = CHUNKS * CH
EP = EPT * NW
PAD_E = EP - E

ROWS = N + 112
RPT = ROWS // NS

CH2 = 80
NBUF2 = 5
CHUNKS2 = E // NW // CH2
KP = 4096
CHK = KP // NW
R1 = 10240
ACC1 = 2 * R1
SPT1 = ACC1 // NS

_MESH = plsc.VectorSubcoreMesh(
    core_axis_name="c", subcore_axis_name="s", num_cores=NC, num_subcores=NS
)


@functools.partial(
    pl.kernel,
    out_type=jax.ShapeDtypeStruct((NC * ACC1,), jnp.float32),
    mesh=_MESH,
    scratch_types=[
        pltpu.VMEM_SHARED((ACC1,), jnp.float32),
        pltpu.VMEM((CHK,), jnp.float32),
        pltpu.VMEM((CHK,), jnp.int32),
    ]
    + [pltpu.VMEM((CH2,), jnp.int32)] * NBUF2
    + [pltpu.SemaphoreType.DMA] * (2 * NBUF2),
)
def _sc_histograms(row_hbm, known_hbm, zeros1_hbm, out_hbm, acc_sh, ones_v,
                   kidx, i0, i1, i2, i3, i4,
                   gi0, gi1, gi2, gi3, gi4, si0, si1, si2, si3, si4):
    ibufs = [i0, i1, i2, i3, i4]
    isems = [gi0, gi1, gi2, gi3, gi4]
    ssems = [si0, si1, si2, si3, si4]
    c = lax.axis_index("c")
    s = lax.axis_index("s")
    wid = s * NC + c
    base = wid * (E // NW)
    pltpu.sync_copy(
        zeros1_hbm.at[pl.ds(s * SPT1, SPT1)], acc_sh.at[pl.ds(s * SPT1, SPT1)]
    )
    for i in range(CHK // 16):
        ones_v[pl.ds(i * 16, 16)] = jnp.full((16,), 1.0, jnp.float32)
    plsc.subcore_barrier()

    pltpu.sync_copy(known_hbm.at[pl.ds(wid * CHK, CHK)], kidx)
    pltpu.sync_copy(ones_v, acc_sh.at[kidx], add=True)

    for b in range(NBUF2):
        pltpu.async_copy(row_hbm.at[pl.ds(base + b * CH2, CH2)], ibufs[b],
                         isems[b])

    def body(r, carry):
        for b in range(NBUF2):
            j = r * NBUF2 + b
            bp = (b - 1) % NBUF2
            pltpu.make_async_copy(row_hbm.at[pl.ds(base + j * CH2, CH2)],
                                  ibufs[b], isems[b]).wait()
            pltpu.async_copy(ones_v.at[pl.ds(0, CH2)], acc_sh.at[ibufs[b]],
                             ssems[b], add=True)

            @pl.when(j >= 1)
            def _():
                pltpu.make_async_copy(ones_v.at[pl.ds(0, CH2)],
                                      acc_sh.at[ibufs[bp]],
                                      ssems[bp]).wait()

                @pl.when(j - 1 + NBUF2 < CHUNKS2)
                def _():
                    pltpu.async_copy(
                        row_hbm.at[pl.ds(base + (j - 1 + NBUF2) * CH2, CH2)],
                        ibufs[bp], isems[bp])
        return carry

    lax.fori_loop(0, CHUNKS2 // NBUF2, body, 0)
    last = (CHUNKS2 - 1) % NBUF2
    pltpu.make_async_copy(ones_v.at[pl.ds(0, CH2)], acc_sh.at[ibufs[last]],
                          ssems[last]).wait()
    plsc.subcore_barrier()
    pltpu.sync_copy(
        acc_sh.at[pl.ds(s * SPT1, SPT1)],
        out_hbm.at[pl.ds(c * ACC1 + s * SPT1, SPT1)],
    )


@functools.partial(
    pl.kernel,
    out_type=jax.ShapeDtypeStruct((NC * ROWS, D), jnp.float32),
    mesh=_MESH,
    scratch_types=[
        pltpu.VMEM_SHARED((ROWS, D), jnp.float32),
        pltpu.VMEM((CHUNKS // NPASS, CH), jnp.int32),
        pltpu.VMEM((CHUNKS // NPASS, CH), jnp.int32),
    ]
    + [pltpu.VMEM((CH, D), jnp.float32)] * NBUF
    + [pltpu.SemaphoreType.DMA] * (2 * NBUF),
)
def _sc_spmm(y_hbm, colp_hbm, rowp_hbm, out_hbm,
             acc_sh, cidx_all, ridx_all, b0, b1, b2, b3,
             g0, g1, g2, g3, s0, s1, s2, s3):
    bufs = [b0, b1, b2, b3]
    gsems = [g0, g1, g2, g3]
    ssems = [s0, s1, s2, s3]
    c = lax.axis_index("c")
    s = lax.axis_index("s")
    wid = s * NC + c
    z16 = jnp.zeros((16,), jnp.float32)
    for r in range(CH):
        for k in range(D // 16):
            b0[r, pl.ds(k * 16, 16)] = z16
    row0 = s * RPT
    for p in range(RPT // CH):
        pltpu.async_copy(b0, acc_sh.at[pl.ds(row0 + p * CH, CH)], g0)
    tail = RPT % CH
    if tail:
        pltpu.async_copy(b0.at[pl.ds(0, tail)],
                         acc_sh.at[pl.ds(row0 + RPT - tail, tail)], g0)
    for p in range(RPT // CH):
        pltpu.make_async_copy(b0, acc_sh.at[pl.ds(row0 + p * CH, CH)],
                              g0).wait()
    if tail:
        pltpu.make_async_copy(b0.at[pl.ds(0, tail)],
                              acc_sh.at[pl.ds(row0 + RPT - tail, tail)],
                              g0).wait()
    plsc.subcore_barrier()

    HC = CHUNKS // NPASS
    for h in range(NPASS):
        pltpu.sync_copy(colp_hbm.at[wid, h], cidx_all)
        pltpu.sync_copy(rowp_hbm.at[wid, h], ridx_all)
        for b in range(NBUF):
            pltpu.async_copy(y_hbm.at[cidx_all.at[b]], bufs[b], gsems[b])

        def round_body(r, carry):
            for b in range(NBUF):
                j = r * NBUF + b
                bp = (b - 1) % NBUF
                pltpu.make_async_copy(y_hbm.at[cidx_all.at[j]], bufs[b],
                                      gsems[b]).wait()
                pltpu.async_copy(bufs[b], acc_sh.at[ridx_all.at[j]], ssems[b],
                                 add=True)
                @pl.when(j >= 1)
                def _():
                    pltpu.make_async_copy(bufs[bp],
                                          acc_sh.at[ridx_all.at[j - 1]],
                                          ssems[bp]).wait()

                    @pl.when(j - 1 + NBUF < HC)
                    def _():
                        pltpu.async_copy(y_hbm.at[cidx_all.at[j - 1 + NBUF]],
                                         bufs[bp], gsems[bp])
            return carry

        lax.fori_loop(0, HC // NBUF, round_body, 0)
        last = (HC - 1) % NBUF
        pltpu.make_async_copy(bufs[last], acc_sh.at[ridx_all.at[HC - 1]],
                              ssems[last]).wait()
    plsc.subcore_barrier()
    out0 = c * ROWS + s * RPT
    npiece = -(-RPT // CH)
    for p in range(npiece):
        nr = min(CH, RPT - p * CH)
        b = bufs[p % NBUF]
        if p >= NBUF:
            prev_nr = min(CH, RPT - (p - NBUF) * CH)
            pltpu.make_async_copy(
                b.at[pl.ds(0, prev_nr)],
                out_hbm.at[pl.ds(out0 + (p - NBUF) * CH, prev_nr)],
                ssems[p % NBUF]).wait()
        pltpu.sync_copy(acc_sh.at[pl.ds(row0 + p * CH, nr)],
                        b.at[pl.ds(0, nr)])
        pltpu.async_copy(b.at[pl.ds(0, nr)],
                         out_hbm.at[pl.ds(out0 + p * CH, nr)],
                         ssems[p % NBUF])
    for p in range(max(0, npiece - NBUF), npiece):
        nr = min(CH, RPT - p * CH)
        pltpu.make_async_copy(bufs[p % NBUF].at[pl.ds(0, nr)],
                              out_hbm.at[pl.ds(out0 + p * CH, nr)],
                              ssems[p % NBUF]).wait()


def _remap_body(r_ref, c_ref, o_ref):
    r = r_ref[...]
    cc = c_ref[...]
    lane = lax.broadcasted_iota(jnp.int32, r.shape, 1)
    o_ref[...] = jnp.where(r == cc, N + (lane % 16), r)


def _tc_remap(row2d, col2d):
    return pl.pallas_call(
        _remap_body,
        out_shape=jax.ShapeDtypeStruct(row2d.shape, jnp.int32),
    )(row2d, col2d)


_BLK = 1264
_NBLK = ROWS // _BLK


def _prep_body(d0_ref, d1_ref, k0_ref, k1_ref, x_ref,
               g2_ref, xk2_ref, g_ref, xk_ref):
    deg = d0_ref[...] + d1_ref[...]
    dis = jnp.where(deg > 0.0, lax.rsqrt(deg), 0.0)
    m = ((k0_ref[...] + k1_ref[...]) > 0.0).astype(jnp.float32)
    disb = jnp.broadcast_to(dis, x_ref.shape)
    g = disb * (1.0 - m)
    xk = x_ref[...] * m
    g2_ref[...] = disb * g
    xk2_ref[...] = disb * xk
    g_ref[...] = g
    xk_ref[...] = xk


def _tc_prep(d0, d1, k0, k1, xpad):
    col_spec = pl.BlockSpec((_BLK, 1), lambda i: (i, 0))
    mat_spec = pl.BlockSpec((_BLK, D), lambda i: (i, 0))
    out_sd = jax.ShapeDtypeStruct((ROWS, D), jnp.float32)
    return pl.pallas_call(
        _prep_body,
        grid=(_NBLK,),
        in_specs=[col_spec, col_spec, col_spec, col_spec, mat_spec],
        out_specs=[mat_spec] * 4,
        out_shape=[out_sd] * 4,
    )(d0, d1, k0, k1, xpad)


def _mid_body(p0_ref, p1_ref, g2_ref, xk2_ref, y_ref):
    y_ref[...] = g2_ref[...] * (p0_ref[...] + p1_ref[...]) + xk2_ref[...]


def _tc_combine_mid(part, g2, xk2):
    p0_spec = pl.BlockSpec((_BLK, D), lambda i: (i, 0))
    p1_spec = pl.BlockSpec((_BLK, D), lambda i: (_NBLK + i, 0))
    mat_spec = pl.BlockSpec((_BLK, D), lambda i: (i, 0))
    return pl.pallas_call(
        _mid_body,
        grid=(_NBLK,),
        in_specs=[p0_spec, p1_spec, mat_spec, mat_spec],
        out_specs=mat_spec,
        out_shape=jax.ShapeDtypeStruct((ROWS, D), jnp.float32),
    )(part, part, g2, xk2)


def _last_body(p0_ref, p1_ref, g_ref, xk_ref, ob_ref):
    ob_ref[...] = g_ref[...] * (p0_ref[...] + p1_ref[...]) + xk_ref[...]


def _tc_combine_last(part, g, xk):
    p0_spec = pl.BlockSpec((_BLK, D), lambda i: (i, 0))
    p1_spec = pl.BlockSpec((_BLK, D), lambda i: (_NBLK + i, 0))
    mat_spec = pl.BlockSpec((_BLK, D), lambda i: (i, 0))
    return pl.pallas_call(
        _last_body,
        grid=(_NBLK,),
        in_specs=[p0_spec, p1_spec, mat_spec, mat_spec],
        out_specs=mat_spec,
        out_shape=jax.ShapeDtypeStruct((ROWS, D), jnp.float32),
    )(part, part, g, xk)


def kernel(x, edge_index, known_feature_mask):
    row = edge_index[0]
    col = edge_index[1]

    pad_i = jnp.arange(PAD_E, dtype=jnp.int32)
    row_pad = jnp.concatenate([row, N + (pad_i % 16)])
    col_pad = jnp.concatenate([col, pad_i % N])
    kpad_i = jnp.arange(KP - K, dtype=jnp.int32)
    known_pad = jnp.concatenate(
        [known_feature_mask + R1, R1 + N + (kpad_i % 16)]
    )
    xpad = jnp.concatenate([x, jnp.zeros((ROWS - N, D), jnp.float32)])

    zeros1 = jnp.zeros((ACC1,), jnp.float32)

    hist = _sc_histograms(row, known_pad, zeros1)
    d0 = hist[0:ROWS].reshape(ROWS, 1)
    d1 = hist[ACC1:ACC1 + ROWS].reshape(ROWS, 1)
    k0 = hist[R1:R1 + ROWS].reshape(ROWS, 1)
    k1 = hist[ACC1 + R1:ACC1 + R1 + ROWS].reshape(ROWS, 1)

    rowp = _tc_remap(
        row_pad.reshape(EP // D, D), col_pad.reshape(EP // D, D)
    ).reshape(EP)

    g2, xk2, g, xk = _tc_prep(d0, d1, k0, k1, xpad)
    y = xk2

    colp3 = col_pad.reshape(NW, NPASS, CHUNKS // NPASS, CH)
    rowp3 = rowp.reshape(NW, NPASS, CHUNKS // NPASS, CH)
    for _ in range(ITERS - 1):
        part = _sc_spmm(y, colp3, rowp3)
        y = _tc_combine_mid(part, g2, xk2)
    part = _sc_spmm(y, colp3, rowp3)
    ob = _tc_combine_last(part, g, xk)
    return ob[0:N]
